# Initial kernel scaffold; baseline (speedup 1.0000x reference)
import jax
import jax.numpy as jnp
from jax.experimental import pallas as pl


def kernel(token_sequence, tok_table, pos_table, ln_w, ln_b):
    raise NotImplementedError("write your pallas kernel here")


# speedup vs baseline: 1.2875x; 1.2875x over previous
import functools

import jax
import jax.numpy as jnp
from jax import lax
from jax.experimental import pallas as pl
from jax.experimental.pallas import tpu as pltpu
from jax.experimental.pallas import tpu_sc as plsc

_VOCAB = 1000000
_NPOS = 256
_D = 64
_B = 4096
_S = 200
_PAD = 0
_EPS = 1e-5

_NW = 32
_CHUNK = 128
_NROWS = _B * _S
_ROWS_PER_W = _NROWS // _NW
_NCHUNKS = _ROWS_PER_W // _CHUNK


def _sc_gather(table, idx3):
    mesh = plsc.VectorSubcoreMesh(core_axis_name="c", subcore_axis_name="s")

    @functools.partial(
        pl.kernel,
        mesh=mesh,
        out_type=jax.ShapeDtypeStruct((_NROWS, _D), jnp.float32),
        compiler_params=pltpu.CompilerParams(use_tc_tiling_on_sc=False),
        scratch_types=[
            pltpu.VMEM((_NCHUNKS, _CHUNK), jnp.int32),
            pltpu.VMEM((_CHUNK, _D), jnp.float32),
            pltpu.SemaphoreType.DMA,
        ],
    )
    def k(table_hbm, idx_hbm, out_hbm, idx_v, rows_v, sem):
        wid = lax.axis_index("s") * 2 + lax.axis_index("c")
        base = wid * _ROWS_PER_W
        pltpu.sync_copy(idx_hbm.at[wid], idx_v)

        def body(c, carry):
            pltpu.async_copy(table_hbm.at[idx_v.at[c]], rows_v, sem).wait()
            pltpu.sync_copy(rows_v, out_hbm.at[pl.ds(base + c * _CHUNK, _CHUNK)])
            return carry

        lax.fori_loop(0, _NCHUNKS, body, 0)

    return k(table, idx3)


_RB = 400


def _ln_body(g_ref, tok_ref, pos_ref, w_ref, b_ref, o_ref):
    g = g_ref[...]
    e = g * (float(_D) ** 0.5) + pos_ref[...]
    mean = jnp.mean(e, axis=1, keepdims=True)
    c = e - mean
    var = jnp.mean(c * c, axis=1, keepdims=True)
    y = c * lax.rsqrt(var + _EPS) * w_ref[...] + b_ref[...]
    o_ref[...] = jnp.where(tok_ref[...] != _PAD, y, 0.0)


def _tc_epilogue(gathered, tok_col, pos2, ln_w, ln_b):
    grid = (_NROWS // _RB,)
    return pl.pallas_call(
        _ln_body,
        grid=grid,
        in_specs=[
            pl.BlockSpec((_RB, _D), lambda i: (i, 0)),
            pl.BlockSpec((_RB, 1), lambda i: (i, 0)),
            pl.BlockSpec((_RB, _D), lambda i: (0, 0)),
            pl.BlockSpec((1, _D), lambda i: (0, 0)),
            pl.BlockSpec((1, _D), lambda i: (0, 0)),
        ],
        out_specs=pl.BlockSpec((_RB, _D), lambda i: (i, 0)),
        out_shape=jax.ShapeDtypeStruct((_NROWS, _D), jnp.float32),
    )(gathered, tok_col, pos2, ln_w, ln_b)


def kernel(token_sequence, tok_table, pos_table, ln_w, ln_b):
    tok = token_sequence.astype(jnp.int32)
    idx3 = tok.reshape(_NW, _NCHUNKS, _CHUNK)
    gathered = _sc_gather(tok_table, idx3)
    pos_rows = lax.slice(pos_table, (1, 0), (_S + 1, _D))
    pos2 = jnp.concatenate([pos_rows, pos_rows], axis=0)
    out = _tc_epilogue(
        gathered,
        tok.reshape(_NROWS, 1),
        pos2,
        ln_w.reshape(1, _D),
        ln_b.reshape(1, _D),
    )
    return out.reshape(_B, _S, _D)


# speedup vs baseline: 2.1851x; 1.6971x over previous
import functools

import jax
import jax.numpy as jnp
from jax import lax
from jax.experimental import pallas as pl
from jax.experimental.pallas import tpu as pltpu
from jax.experimental.pallas import tpu_sc as plsc

_VOCAB = 1000000
_NPOS = 256
_D = 64
_B = 4096
_S = 200
_PAD = 0
_EPS = 1e-5

_NW = 32
_CHUNK = 128
_NROWS = _B * _S
_ROWS_PER_W = _NROWS // _NW
_NCHUNKS = _ROWS_PER_W // _CHUNK

_RB = 3200


def _sc_gather(table, idx3):
    mesh = plsc.VectorSubcoreMesh(core_axis_name="c", subcore_axis_name="s")

    @functools.partial(
        pl.kernel,
        mesh=mesh,
        out_type=jax.ShapeDtypeStruct((_NROWS, _D), jnp.float32),
        compiler_params=pltpu.CompilerParams(
            use_tc_tiling_on_sc=False, needs_layout_passes=False),
        scratch_types=[
            pltpu.VMEM((_NCHUNKS, _CHUNK), jnp.int32),
            pltpu.VMEM((_CHUNK, _D), jnp.float32),
            pltpu.VMEM((_CHUNK, _D), jnp.float32),
            pltpu.SemaphoreType.DMA,
            pltpu.SemaphoreType.DMA,
        ],
    )
    def k(table_hbm, idx_hbm, out_hbm, idx_v, rows0, rows1, sem0, sem1):
        wid = lax.axis_index("s") * 2 + lax.axis_index("c")
        base = wid * _ROWS_PER_W
        pltpu.sync_copy(idx_hbm.at[wid], idx_v)

        def gather(c, buf, sem):
            return pltpu.make_async_copy(table_hbm.at[idx_v.at[c]], buf, sem)

        gather(0, rows0, sem0).start()
        gather(1, rows1, sem1).start()

        inf16 = jnp.full((16,), jnp.inf, dtype=jnp.float32)
        zeros16 = jnp.zeros((16,), dtype=jnp.int32)
        iota16 = lax.iota(jnp.int32, 16)

        def step(c, buf, sem):
            gather(c, buf, sem).wait()
            for g in range(_CHUNK // 16):
                tokv = idx_v[c, pl.ds(g * 16, 16)]
                plsc.store_scatter(
                    buf, [iota16 + (g * 16), zeros16], inf16,
                    mask=tokv == _PAD)
            pltpu.sync_copy(buf, out_hbm.at[pl.ds(base + c * _CHUNK, _CHUNK)])

            @pl.when(c + 2 < _NCHUNKS)
            def _():
                gather(c + 2, buf, sem).start()

        def body(i, carry):
            step(2 * i, rows0, sem0)
            step(2 * i + 1, rows1, sem1)
            return carry

        lax.fori_loop(0, _NCHUNKS // 2, body, 0)

    return k(table, idx3)


def _ln_body(g_ref, pos_ref, w_ref, b_ref, o_ref):
    g = g_ref[...]
    valid = g_ref[:, 0:1] != jnp.inf
    e = g * (float(_D) ** 0.5) + pos_ref[...]
    mean = jnp.mean(e, axis=1, keepdims=True)
    c = e - mean
    var = jnp.mean(c * c, axis=1, keepdims=True)
    y = c * lax.rsqrt(var + _EPS) * w_ref[...] + b_ref[...]
    o_ref[...] = jnp.where(valid, y, 0.0)


def _tc_epilogue(gathered, pos_tiled, ln_w, ln_b):
    grid = (_NROWS // _RB,)
    return pl.pallas_call(
        _ln_body,
        grid=grid,
        in_specs=[
            pl.BlockSpec((_RB, _D), lambda i: (i, 0)),
            pl.BlockSpec((_RB, _D), lambda i: (0, 0)),
            pl.BlockSpec((1, _D), lambda i: (0, 0)),
            pl.BlockSpec((1, _D), lambda i: (0, 0)),
        ],
        out_specs=pl.BlockSpec((_RB, _D), lambda i: (i, 0)),
        out_shape=jax.ShapeDtypeStruct((_NROWS, _D), jnp.float32),
    )(gathered, pos_tiled, ln_w, ln_b)


def kernel(token_sequence, tok_table, pos_table, ln_w, ln_b):
    tok = token_sequence.astype(jnp.int32)
    idx3 = tok.reshape(_NW, _NCHUNKS, _CHUNK)
    gathered = _sc_gather(tok_table, idx3)
    pos_rows = lax.slice(pos_table, (1, 0), (_S + 1, _D))
    pos_tiled = jnp.tile(pos_rows, (_RB // _S, 1))
    out = _tc_epilogue(
        gathered,
        pos_tiled,
        ln_w.reshape(1, _D),
        ln_b.reshape(1, _D),
    )
    return out.reshape(_B, _S, _D)
